# Initial kernel scaffold; baseline (speedup 1.0000x reference)
import jax
import jax.numpy as jnp
from jax.experimental import pallas as pl


def kernel(points, W1, b1, W2, b2, W3, b3, W4, b4, Wd1, bd1, Wd2, bd2, Wd3, bd3, Wd4, bd4):
    raise NotImplementedError("write your pallas kernel here")


# speedup vs baseline: 3.9109x; 3.9109x over previous
import functools

import jax
import jax.numpy as jnp
from jax.experimental import pallas as pl
from jax.experimental.pallas import tpu as pltpu

N_POINTS = 16384
N1 = 819
N2 = 40
K = 32
R1 = 0.3
R2 = 1.0


def _fps_body(n_samples, x_ref, y_ref, z_ref, inds_ref):
    x = x_ref[...]
    y = y_ref[...]
    z = z_ref[...]
    shape = x.shape
    rows = jax.lax.broadcasted_iota(jnp.int32, shape, 0)
    cols = jax.lax.broadcasted_iota(jnp.int32, shape, 1)
    flat = rows * shape[1] + cols

    px = x[0, 0]
    py = y[0, 0]
    pz = z[0, 0]
    d = (x - px) ** 2 + (y - py) ** 2 + (z - pz) ** 2
    inds_ref[0] = 0

    def body(i, d):
        m = jnp.max(d)
        idx = jnp.min(jnp.where(d == m, flat, jnp.int32(2**30)))
        inds_ref[i] = idx
        onehot = (flat == idx).astype(jnp.float32)
        px = jnp.sum(x * onehot)
        py = jnp.sum(y * onehot)
        pz = jnp.sum(z * onehot)
        d = jnp.minimum(d, (x - px) ** 2 + (y - py) ** 2 + (z - pz) ** 2)
        return d

    jax.lax.fori_loop(1, n_samples, body, d)


def _fps(points_T, n_samples, n_rows):
    return pl.pallas_call(
        functools.partial(_fps_body, n_samples),
        out_shape=jax.ShapeDtypeStruct((n_samples,), jnp.int32),
        in_specs=[
            pl.BlockSpec(memory_space=pltpu.MemorySpace.VMEM),
            pl.BlockSpec(memory_space=pltpu.MemorySpace.VMEM),
            pl.BlockSpec(memory_space=pltpu.MemorySpace.VMEM),
        ],
        out_specs=pl.BlockSpec(memory_space=pltpu.MemorySpace.SMEM),
    )(points_T[0], points_T[1], points_T[2])


def _radius_knn(points, samples, r, k):
    d2 = jnp.sum((samples[:, None, :] - points[None, :, :]) ** 2, axis=-1)
    negd, idx = jax.lax.top_k(-d2, k)
    valid = (-negd) <= r * r
    return idx.reshape(-1).astype(jnp.int32), valid.reshape(-1)


def kernel(points, W1, b1, W2, b2, W3, b3, W4, b4,
           Wd1, bd1, Wd2, bd2, Wd3, bd3, Wd4, bd4):
    pts_T = points.T.reshape(3, 128, 128)
    s_inds = _fps(pts_T, N1, 128)
    samples = points[s_inds]

    id1, v1 = _radius_knn(points, samples, R1, K)
    rad_points = points[id1]
    midpoints = jnp.repeat(samples, K, axis=0)
    relative = (rad_points - midpoints) / R1 * v1[:, None].astype(points.dtype)

    s2_inds = _fps_masked(samples, N2)
    samples2 = samples[s2_inds]

    id2, v2 = _radius_knn(samples, samples2, R2, K)
    rad2_points = samples[id2]
    midpoints2 = jnp.repeat(samples2, K, axis=0)
    relative2 = (rad2_points - midpoints2) / R2 * v2[:, None].astype(points.dtype)

    h1 = jax.nn.relu(relative @ W1 + b1)
    h1 = jax.nn.relu(h1 @ W2 + b2)
    feats = jnp.max(h1.reshape(N1, K, -1), axis=1)

    inp2 = jnp.concatenate([relative2, feats[id2]], axis=1)
    h2 = jax.nn.relu(inp2 @ W3 + b3)
    h2 = jax.nn.relu(h2 @ W4 + b4)
    encoding = jnp.max(h2.reshape(N2, K, -1), axis=1)

    gfeat = jnp.max(feats, axis=0)
    dec_in = jnp.concatenate([encoding, jnp.tile(gfeat[None, :], (N2, 1))], axis=1)
    d1 = jax.nn.relu(dec_in @ Wd1 + bd1)
    mid_feats = (d1 @ Wd2 + bd2).reshape(N2 * 20, 64)
    decoded = mid_feats @ Wd3 + bd3
    decoded2 = (mid_feats @ Wd4 + bd4).reshape(N2 * 20 * 20, 3)

    midpoints_out = (jnp.repeat(samples2, 20, axis=0) + decoded) * R2
    points_out = (jnp.repeat(midpoints_out, 20, axis=0) + decoded2) * R1
    return points_out


def _fps_masked_body(n_samples, n_valid, x_ref, y_ref, z_ref, inds_ref):
    x = x_ref[...]
    y = y_ref[...]
    z = z_ref[...]
    shape = x.shape
    rows = jax.lax.broadcasted_iota(jnp.int32, shape, 0)
    cols = jax.lax.broadcasted_iota(jnp.int32, shape, 1)
    flat = rows * shape[1] + cols
    invalid = flat >= n_valid
    neginf = jnp.float32(-jnp.inf)

    px = x[0, 0]
    py = y[0, 0]
    pz = z[0, 0]
    d = (x - px) ** 2 + (y - py) ** 2 + (z - pz) ** 2
    d = jnp.where(invalid, neginf, d)
    inds_ref[0] = 0

    def body(i, d):
        m = jnp.max(d)
        idx = jnp.min(jnp.where(d == m, flat, jnp.int32(2**30)))
        inds_ref[i] = idx
        onehot = (flat == idx).astype(jnp.float32)
        px = jnp.sum(x * onehot)
        py = jnp.sum(y * onehot)
        pz = jnp.sum(z * onehot)
        d = jnp.minimum(d, (x - px) ** 2 + (y - py) ** 2 + (z - pz) ** 2)
        return d

    jax.lax.fori_loop(1, n_samples, body, d)


def _fps_masked(samples, n_samples):
    n = samples.shape[0]
    pad = (-n) % 1024
    sp = jnp.pad(samples, ((0, pad), (0, 0)))
    sp_T = sp.T.reshape(3, -1, 128)
    return pl.pallas_call(
        functools.partial(_fps_masked_body, n_samples, n),
        out_shape=jax.ShapeDtypeStruct((n_samples,), jnp.int32),
        in_specs=[
            pl.BlockSpec(memory_space=pltpu.MemorySpace.VMEM),
            pl.BlockSpec(memory_space=pltpu.MemorySpace.VMEM),
            pl.BlockSpec(memory_space=pltpu.MemorySpace.VMEM),
        ],
        out_specs=pl.BlockSpec(memory_space=pltpu.MemorySpace.SMEM),
    )(sp_T[0], sp_T[1], sp_T[2])


# speedup vs baseline: 3.9431x; 1.0082x over previous
import functools

import jax
import jax.numpy as jnp
from jax.experimental import pallas as pl
from jax.experimental.pallas import tpu as pltpu

N_POINTS = 16384
N1 = 819
N2 = 40
K = 32
R1 = 0.3
R2 = 1.0


def _fps_body(n_samples, n_valid, x_ref, y_ref, z_ref, inds_ref):
    x = x_ref[...]
    y = y_ref[...]
    z = z_ref[...]
    shape = x.shape
    rows = jax.lax.broadcasted_iota(jnp.int32, shape, 0)
    cols = jax.lax.broadcasted_iota(jnp.int32, shape, 1)
    flat = rows * shape[1] + cols

    px = x[0, 0]
    py = y[0, 0]
    pz = z[0, 0]
    d = (x - px) ** 2 + (y - py) ** 2 + (z - pz) ** 2
    if n_valid < shape[0] * shape[1]:
        d = jnp.where(flat >= n_valid, jnp.float32(-jnp.inf), d)
    inds_ref[0] = 0

    def body(i, d):
        m = jnp.max(d)
        idx = jnp.min(jnp.where(d == m, flat, jnp.int32(2**30)))
        inds_ref[i] = idx
        onehot = (flat == idx).astype(jnp.float32)
        px = jnp.sum(x * onehot)
        py = jnp.sum(y * onehot)
        pz = jnp.sum(z * onehot)
        d = jnp.minimum(d, (x - px) ** 2 + (y - py) ** 2 + (z - pz) ** 2)
        return d

    jax.lax.fori_loop(1, n_samples, body, d)


def _fps(points, n_samples):
    n = points.shape[0]
    pad = (-n) % 1024
    sp = jnp.pad(points, ((0, pad), (0, 0)))
    sp_T = sp.T.reshape(3, -1, 128)
    return pl.pallas_call(
        functools.partial(_fps_body, n_samples, n),
        out_shape=jax.ShapeDtypeStruct((n_samples,), jnp.int32),
        in_specs=[
            pl.BlockSpec(memory_space=pltpu.MemorySpace.VMEM),
            pl.BlockSpec(memory_space=pltpu.MemorySpace.VMEM),
            pl.BlockSpec(memory_space=pltpu.MemorySpace.VMEM),
        ],
        out_specs=pl.BlockSpec(memory_space=pltpu.MemorySpace.SMEM),
    )(sp_T[0], sp_T[1], sp_T[2])


_QB = 8


def _topk_body(x_ref, y_ref, z_ref, q_ref, ids_ref, vals_ref):
    n = x_ref.shape[1]
    qx = q_ref[:, 0:1]
    qy = q_ref[:, 1:2]
    qz = q_ref[:, 2:3]
    d2 = (x_ref[...] - qx) ** 2 + (y_ref[...] - qy) ** 2 + (z_ref[...] - qz) ** 2
    flat = jax.lax.broadcasted_iota(jnp.int32, (_QB, n), 1)
    kcol = jax.lax.broadcasted_iota(jnp.int32, (_QB, K), 1)
    vals0 = jnp.zeros((_QB, K), jnp.float32)
    ids0 = jnp.zeros((_QB, K), jnp.int32)

    def body(r, carry):
        d2, vals, ids = carry
        m = jnp.min(d2, axis=1, keepdims=True)
        idx = jnp.min(jnp.where(d2 == m, flat, jnp.int32(2**30)),
                      axis=1, keepdims=True)
        vals = jnp.where(kcol == r, m, vals)
        ids = jnp.where(kcol == r, idx, ids)
        d2 = jnp.where(flat == idx, jnp.float32(jnp.inf), d2)
        return d2, vals, ids

    _, vals, ids = jax.lax.fori_loop(0, K, body, (d2, vals0, ids0))
    ids_ref[...] = ids
    vals_ref[...] = vals


def _topk(points, samples):
    s = samples.shape[0]
    spad = (-s) % _QB
    sp = jnp.pad(samples, ((0, spad), (0, 0)))
    nq = sp.shape[0]
    npad = (-points.shape[0]) % 128
    pp = jnp.pad(points, ((0, npad), (0, 0)), constant_values=1e6)
    n = pp.shape[0]
    planes = jnp.broadcast_to(pp.T[:, None, :], (3, _QB, n))
    grid = nq // _QB
    ids, vals = pl.pallas_call(
        _topk_body,
        grid=(grid,),
        in_specs=[
            pl.BlockSpec((_QB, n), lambda i: (0, 0)),
            pl.BlockSpec((_QB, n), lambda i: (0, 0)),
            pl.BlockSpec((_QB, n), lambda i: (0, 0)),
            pl.BlockSpec((_QB, 3), lambda i: (i, 0)),
        ],
        out_specs=[
            pl.BlockSpec((_QB, K), lambda i: (i, 0)),
            pl.BlockSpec((_QB, K), lambda i: (i, 0)),
        ],
        out_shape=[
            jax.ShapeDtypeStruct((nq, K), jnp.int32),
            jax.ShapeDtypeStruct((nq, K), jnp.float32),
        ],
    )(planes[0], planes[1], planes[2], sp)
    return ids[:s], vals[:s]


def kernel(points, W1, b1, W2, b2, W3, b3, W4, b4,
           Wd1, bd1, Wd2, bd2, Wd3, bd3, Wd4, bd4):
    s_inds = _fps(points, N1)
    samples = points[s_inds]

    ids1, dv1 = _topk(points, samples)
    id1 = ids1.reshape(-1)
    v1 = (dv1 <= R1 * R1).reshape(-1)
    rad_points = points[id1]
    midpoints = jnp.repeat(samples, K, axis=0)
    relative = (rad_points - midpoints) / R1 * v1[:, None].astype(points.dtype)

    s2_inds = _fps(samples, N2)
    samples2 = samples[s2_inds]

    ids2, dv2 = _topk(samples, samples2)
    id2 = ids2.reshape(-1)
    v2 = (dv2 <= R2 * R2).reshape(-1)
    rad2_points = samples[id2]
    midpoints2 = jnp.repeat(samples2, K, axis=0)
    relative2 = (rad2_points - midpoints2) / R2 * v2[:, None].astype(points.dtype)

    h1 = jax.nn.relu(relative @ W1 + b1)
    h1 = jax.nn.relu(h1 @ W2 + b2)
    feats = jnp.max(h1.reshape(N1, K, -1), axis=1)

    inp2 = jnp.concatenate([relative2, feats[id2]], axis=1)
    h2 = jax.nn.relu(inp2 @ W3 + b3)
    h2 = jax.nn.relu(h2 @ W4 + b4)
    encoding = jnp.max(h2.reshape(N2, K, -1), axis=1)

    gfeat = jnp.max(feats, axis=0)
    dec_in = jnp.concatenate([encoding, jnp.tile(gfeat[None, :], (N2, 1))], axis=1)
    d1 = jax.nn.relu(dec_in @ Wd1 + bd1)
    mid_feats = (d1 @ Wd2 + bd2).reshape(N2 * 20, 64)
    decoded = mid_feats @ Wd3 + bd3
    decoded2 = (mid_feats @ Wd4 + bd4).reshape(N2 * 20 * 20, 3)

    midpoints_out = (jnp.repeat(samples2, 20, axis=0) + decoded) * R2
    points_out = (jnp.repeat(midpoints_out, 20, axis=0) + decoded2) * R1
    return points_out


# speedup vs baseline: 3.9508x; 1.0020x over previous
import functools

import jax
import jax.numpy as jnp
from jax.experimental import pallas as pl
from jax.experimental.pallas import tpu as pltpu

N_POINTS = 16384
N1 = 819
N2 = 40
K = 32
R1 = 0.3
R2 = 1.0


def _fps_body(n_samples, n_valid, x_ref, y_ref, z_ref, inds_ref):
    x = x_ref[...]
    y = y_ref[...]
    z = z_ref[...]
    shape = x.shape
    rows = jax.lax.broadcasted_iota(jnp.int32, shape, 0)
    cols = jax.lax.broadcasted_iota(jnp.int32, shape, 1)
    flat = rows * shape[1] + cols

    px = x[0, 0]
    py = y[0, 0]
    pz = z[0, 0]
    d = (x - px) ** 2 + (y - py) ** 2 + (z - pz) ** 2
    if n_valid < shape[0] * shape[1]:
        d = jnp.where(flat >= n_valid, jnp.float32(-jnp.inf), d)
    inds_ref[0] = 0

    def body(i, d):
        m = jnp.max(d)
        idx = jnp.min(jnp.where(d == m, flat, jnp.int32(2**30)))
        inds_ref[i] = idx
        onehot = (flat == idx).astype(jnp.float32)
        px = jnp.sum(x * onehot)
        py = jnp.sum(y * onehot)
        pz = jnp.sum(z * onehot)
        d = jnp.minimum(d, (x - px) ** 2 + (y - py) ** 2 + (z - pz) ** 2)
        return d

    jax.lax.fori_loop(1, n_samples, body, d)


def _fps(points, n_samples):
    n = points.shape[0]
    pad = (-n) % 1024
    sp = jnp.pad(points, ((0, pad), (0, 0)))
    sp_T = sp.T.reshape(3, -1, 128)
    return pl.pallas_call(
        functools.partial(_fps_body, n_samples, n),
        out_shape=jax.ShapeDtypeStruct((n_samples,), jnp.int32),
        in_specs=[
            pl.BlockSpec(memory_space=pltpu.MemorySpace.VMEM),
            pl.BlockSpec(memory_space=pltpu.MemorySpace.VMEM),
            pl.BlockSpec(memory_space=pltpu.MemorySpace.VMEM),
        ],
        out_specs=pl.BlockSpec(memory_space=pltpu.MemorySpace.SMEM),
    )(sp_T[0], sp_T[1], sp_T[2])


_QB = 8


def _topk_body(x_ref, y_ref, z_ref, q_ref, ids_ref, vals_ref):
    n = x_ref.shape[1]
    qx = q_ref[:, 0:1]
    qy = q_ref[:, 1:2]
    qz = q_ref[:, 2:3]
    d2 = (x_ref[...] - qx) ** 2 + (y_ref[...] - qy) ** 2 + (z_ref[...] - qz) ** 2
    flat = jax.lax.broadcasted_iota(jnp.int32, (_QB, n), 1)
    kcol = jax.lax.broadcasted_iota(jnp.int32, (_QB, K), 1)
    vals0 = jnp.zeros((_QB, K), jnp.float32)
    ids0 = jnp.zeros((_QB, K), jnp.int32)

    def body(r, carry):
        d2, vals, ids = carry
        m = jnp.min(d2, axis=1, keepdims=True)
        idx = jnp.min(jnp.where(d2 == m, flat, jnp.int32(2**30)),
                      axis=1, keepdims=True)
        vals = jnp.where(kcol == r, m, vals)
        ids = jnp.where(kcol == r, idx, ids)
        d2 = jnp.where(flat == idx, jnp.float32(jnp.inf), d2)
        return d2, vals, ids

    _, vals, ids = jax.lax.fori_loop(0, K, body, (d2, vals0, ids0))
    ids_ref[...] = ids
    vals_ref[...] = vals


_G = 128
_NGSEL = 36


def _topk_big_body(x_ref, y_ref, z_ref, q_ref, ids_ref, vals_ref,
                   d2_ref, cand_ref, candf_ref):
    n = x_ref.shape[1]
    ng = n // _G
    qx = q_ref[:, 0:1]
    qy = q_ref[:, 1:2]
    qz = q_ref[:, 2:3]
    d2 = (x_ref[...] - qx) ** 2 + (y_ref[...] - qy) ** 2 + (z_ref[...] - qz) ** 2
    d2_ref[...] = d2

    gm = jnp.concatenate(
        [jnp.min(d2[:, g * _G:(g + 1) * _G], axis=1, keepdims=True)
         for g in range(ng)], axis=1)
    giota = jax.lax.broadcasted_iota(jnp.int32, (_QB, ng), 1)
    scol = jax.lax.broadcasted_iota(jnp.int32, (_QB, _NGSEL), 1)

    def selbody(r, carry):
        gm, gl = carry
        m = jnp.min(gm, axis=1, keepdims=True)
        g = jnp.min(jnp.where(gm == m, giota, jnp.int32(ng)),
                    axis=1, keepdims=True)
        gl = jnp.where(scol == r, g, gl)
        gm = jnp.where(giota == g, jnp.float32(jnp.inf), gm)
        return gm, gl

    _, gl = jax.lax.fori_loop(
        0, _NGSEL, selbody, (gm, jnp.zeros((_QB, _NGSEL), jnp.int32)))

    lane = jax.lax.broadcasted_iota(jnp.int32, (1, _G), 1)
    for r in range(_NGSEL):
        for q in range(_QB):
            start = gl[q, r] * _G
            cand_ref[pl.ds(q, 1), pl.ds(r * _G, _G)] = (
                d2_ref[pl.ds(q, 1), pl.ds(start, _G)])
            candf_ref[pl.ds(q, 1), pl.ds(r * _G, _G)] = start + lane

    cand0 = cand_ref[...]
    candf = candf_ref[...]
    kcol = jax.lax.broadcasted_iota(jnp.int32, (_QB, K), 1)

    def exbody(r, carry):
        cand, vals, ids = carry
        m = jnp.min(cand, axis=1, keepdims=True)
        idx = jnp.min(jnp.where(cand == m, candf, jnp.int32(2**30)),
                      axis=1, keepdims=True)
        vals = jnp.where(kcol == r, m, vals)
        ids = jnp.where(kcol == r, idx, ids)
        cand = jnp.where(candf == idx, jnp.float32(jnp.inf), cand)
        return cand, vals, ids

    _, vals, ids = jax.lax.fori_loop(
        0, K, exbody, (cand0, jnp.zeros((_QB, K), jnp.float32),
                       jnp.zeros((_QB, K), jnp.int32)))
    ids_ref[...] = ids
    vals_ref[...] = vals


def _topk_big(points, samples):
    s = samples.shape[0]
    spad = (-s) % _QB
    sp = jnp.pad(samples, ((0, spad), (0, 0)))
    nq = sp.shape[0]
    n = points.shape[0]
    planes = jnp.broadcast_to(points.T[:, None, :], (3, _QB, n))
    grid = nq // _QB
    ids, vals = pl.pallas_call(
        _topk_big_body,
        grid=(grid,),
        in_specs=[
            pl.BlockSpec((_QB, n), lambda i: (0, 0)),
            pl.BlockSpec((_QB, n), lambda i: (0, 0)),
            pl.BlockSpec((_QB, n), lambda i: (0, 0)),
            pl.BlockSpec((_QB, 3), lambda i: (i, 0)),
        ],
        out_specs=[
            pl.BlockSpec((_QB, K), lambda i: (i, 0)),
            pl.BlockSpec((_QB, K), lambda i: (i, 0)),
        ],
        out_shape=[
            jax.ShapeDtypeStruct((nq, K), jnp.int32),
            jax.ShapeDtypeStruct((nq, K), jnp.float32),
        ],
        scratch_shapes=[
            pltpu.VMEM((_QB, n), jnp.float32),
            pltpu.VMEM((_QB, _NGSEL * _G), jnp.float32),
            pltpu.VMEM((_QB, _NGSEL * _G), jnp.int32),
        ],
    )(planes[0], planes[1], planes[2], sp)
    return ids[:s], vals[:s]


def _topk(points, samples):
    s = samples.shape[0]
    spad = (-s) % _QB
    sp = jnp.pad(samples, ((0, spad), (0, 0)))
    nq = sp.shape[0]
    npad = (-points.shape[0]) % 128
    pp = jnp.pad(points, ((0, npad), (0, 0)), constant_values=1e6)
    n = pp.shape[0]
    planes = jnp.broadcast_to(pp.T[:, None, :], (3, _QB, n))
    grid = nq // _QB
    ids, vals = pl.pallas_call(
        _topk_body,
        grid=(grid,),
        in_specs=[
            pl.BlockSpec((_QB, n), lambda i: (0, 0)),
            pl.BlockSpec((_QB, n), lambda i: (0, 0)),
            pl.BlockSpec((_QB, n), lambda i: (0, 0)),
            pl.BlockSpec((_QB, 3), lambda i: (i, 0)),
        ],
        out_specs=[
            pl.BlockSpec((_QB, K), lambda i: (i, 0)),
            pl.BlockSpec((_QB, K), lambda i: (i, 0)),
        ],
        out_shape=[
            jax.ShapeDtypeStruct((nq, K), jnp.int32),
            jax.ShapeDtypeStruct((nq, K), jnp.float32),
        ],
    )(planes[0], planes[1], planes[2], sp)
    return ids[:s], vals[:s]


def kernel(points, W1, b1, W2, b2, W3, b3, W4, b4,
           Wd1, bd1, Wd2, bd2, Wd3, bd3, Wd4, bd4):
    s_inds = _fps(points, N1)
    samples = points[s_inds]

    ids1, dv1 = _topk_big(points, samples)
    id1 = ids1.reshape(-1)
    v1 = (dv1 <= R1 * R1).reshape(-1)
    rad_points = points[id1]
    midpoints = jnp.repeat(samples, K, axis=0)
    relative = (rad_points - midpoints) / R1 * v1[:, None].astype(points.dtype)

    s2_inds = _fps(samples, N2)
    samples2 = samples[s2_inds]

    ids2, dv2 = _topk(samples, samples2)
    id2 = ids2.reshape(-1)
    v2 = (dv2 <= R2 * R2).reshape(-1)
    rad2_points = samples[id2]
    midpoints2 = jnp.repeat(samples2, K, axis=0)
    relative2 = (rad2_points - midpoints2) / R2 * v2[:, None].astype(points.dtype)

    h1 = jax.nn.relu(relative @ W1 + b1)
    h1 = jax.nn.relu(h1 @ W2 + b2)
    feats = jnp.max(h1.reshape(N1, K, -1), axis=1)

    inp2 = jnp.concatenate([relative2, feats[id2]], axis=1)
    h2 = jax.nn.relu(inp2 @ W3 + b3)
    h2 = jax.nn.relu(h2 @ W4 + b4)
    encoding = jnp.max(h2.reshape(N2, K, -1), axis=1)

    gfeat = jnp.max(feats, axis=0)
    dec_in = jnp.concatenate([encoding, jnp.tile(gfeat[None, :], (N2, 1))], axis=1)
    d1 = jax.nn.relu(dec_in @ Wd1 + bd1)
    mid_feats = (d1 @ Wd2 + bd2).reshape(N2 * 20, 64)
    decoded = mid_feats @ Wd3 + bd3
    decoded2 = (mid_feats @ Wd4 + bd4).reshape(N2 * 20 * 20, 3)

    midpoints_out = (jnp.repeat(samples2, 20, axis=0) + decoded) * R2
    points_out = (jnp.repeat(midpoints_out, 20, axis=0) + decoded2) * R1
    return points_out


# speedup vs baseline: 8.3479x; 2.1130x over previous
import functools

import jax
import jax.numpy as jnp
from jax.experimental import pallas as pl
from jax.experimental.pallas import tpu as pltpu

N_POINTS = 16384
N1 = 819
N2 = 40
K = 32
R1 = 0.3
R2 = 1.0


def _fps_body(n_samples, n_valid, x_ref, y_ref, z_ref, inds_ref):
    x = x_ref[...]
    y = y_ref[...]
    z = z_ref[...]
    shape = x.shape
    rows = jax.lax.broadcasted_iota(jnp.int32, shape, 0)
    cols = jax.lax.broadcasted_iota(jnp.int32, shape, 1)
    flat = rows * shape[1] + cols

    px = x[0, 0]
    py = y[0, 0]
    pz = z[0, 0]
    d = (x - px) ** 2 + (y - py) ** 2 + (z - pz) ** 2
    if n_valid < shape[0] * shape[1]:
        d = jnp.where(flat >= n_valid, jnp.float32(-jnp.inf), d)
    inds_ref[0] = 0

    def body(i, d):
        m = jnp.max(d)
        idx = jnp.min(jnp.where(d == m, flat, jnp.int32(2**30)))
        inds_ref[i] = idx
        onehot = (flat == idx).astype(jnp.float32)
        px = jnp.sum(x * onehot)
        py = jnp.sum(y * onehot)
        pz = jnp.sum(z * onehot)
        d = jnp.minimum(d, (x - px) ** 2 + (y - py) ** 2 + (z - pz) ** 2)
        return d

    jax.lax.fori_loop(1, n_samples, body, d)


def _fps(points, n_samples):
    n = points.shape[0]
    pad = (-n) % 1024
    sp = jnp.pad(points, ((0, pad), (0, 0)))
    sp_T = sp.T.reshape(3, -1, 128)
    return pl.pallas_call(
        functools.partial(_fps_body, n_samples, n),
        out_shape=jax.ShapeDtypeStruct((n_samples,), jnp.int32),
        in_specs=[
            pl.BlockSpec(memory_space=pltpu.MemorySpace.VMEM),
            pl.BlockSpec(memory_space=pltpu.MemorySpace.VMEM),
            pl.BlockSpec(memory_space=pltpu.MemorySpace.VMEM),
        ],
        out_specs=pl.BlockSpec(memory_space=pltpu.MemorySpace.SMEM),
    )(sp_T[0], sp_T[1], sp_T[2])


_QB = 8


def _topk_body(x_ref, y_ref, z_ref, q_ref, ids_ref, vals_ref):
    n = x_ref.shape[1]
    qx = q_ref[:, 0:1]
    qy = q_ref[:, 1:2]
    qz = q_ref[:, 2:3]
    d2 = (x_ref[...] - qx) ** 2 + (y_ref[...] - qy) ** 2 + (z_ref[...] - qz) ** 2
    flat = jax.lax.broadcasted_iota(jnp.int32, (_QB, n), 1)
    kcol = jax.lax.broadcasted_iota(jnp.int32, (_QB, K), 1)
    vals0 = jnp.zeros((_QB, K), jnp.float32)
    ids0 = jnp.zeros((_QB, K), jnp.int32)

    def body(r, carry):
        d2, vals, ids = carry
        m = jnp.min(d2, axis=1, keepdims=True)
        idx = jnp.min(jnp.where(d2 == m, flat, jnp.int32(2**30)),
                      axis=1, keepdims=True)
        vals = jnp.where(kcol == r, m, vals)
        ids = jnp.where(kcol == r, idx, ids)
        d2 = jnp.where(flat == idx, jnp.float32(jnp.inf), d2)
        return d2, vals, ids

    _, vals, ids = jax.lax.fori_loop(0, K, body, (d2, vals0, ids0))
    ids_ref[...] = ids
    vals_ref[...] = vals


_G = 128
_QBB = 32


def _fold_min(v, w):
    while w > _G:
        h = w // 2
        v = jnp.minimum(v[:, :h], v[:, h:])
        w = h
    return jnp.min(v, axis=1, keepdims=True)


def _lex_min(cand, candf):
    m = _fold_min(cand, cand.shape[1])
    iv = jnp.where(cand == m, candf, jnp.int32(2**30))
    while iv.shape[1] > _G:
        h = iv.shape[1] // 2
        iv = jnp.minimum(iv[:, :h], iv[:, h:])
    idx = jnp.min(iv, axis=1, keepdims=True)
    return m, idx


def _topk_big_body(x_ref, y_ref, z_ref, q_ref, ids_ref, vals_ref,
                   d2_ref, cand_ref, candf_ref):
    n = x_ref.shape[1]
    ng = n // _G
    qx = q_ref[:, 0:1]
    qy = q_ref[:, 1:2]
    qz = q_ref[:, 2:3]
    d2 = (x_ref[...] - qx) ** 2 + (y_ref[...] - qy) ** 2 + (z_ref[...] - qz) ** 2
    d2_ref[...] = d2

    gm = jnp.concatenate(
        [jnp.min(d2[:, g * _G:(g + 1) * _G], axis=1, keepdims=True)
         for g in range(ng)], axis=1)
    giota = jax.lax.broadcasted_iota(jnp.int32, (_QBB, ng), 1)
    scol = jax.lax.broadcasted_iota(jnp.int32, (_QBB, K), 1)

    def selbody(r, carry):
        gm, gl = carry
        m = jnp.min(gm, axis=1, keepdims=True)
        g = jnp.min(jnp.where(gm == m, giota, jnp.int32(ng)),
                    axis=1, keepdims=True)
        gl = jnp.where(scol == r, g, gl)
        gm = jnp.where(giota == g, jnp.float32(jnp.inf), gm)
        return gm, gl

    _, gl = jax.lax.fori_loop(
        0, K, selbody, (gm, jnp.zeros((_QBB, K), jnp.int32)))

    lane = jax.lax.broadcasted_iota(jnp.int32, (1, _G), 1)
    for r in range(K):
        for q in range(_QBB):
            start = gl[q, r] * _G
            cand_ref[pl.ds(q, 1), pl.ds(r * _G, _G)] = (
                d2_ref[pl.ds(q, 1), pl.ds(start, _G)])
            candf_ref[pl.ds(q, 1), pl.ds(r * _G, _G)] = start + lane

    cand0 = cand_ref[...]
    candf = candf_ref[...]
    kcol = jax.lax.broadcasted_iota(jnp.int32, (_QBB, K), 1)

    def exbody(r, carry):
        cand, vals, ids = carry
        m, idx = _lex_min(cand, candf)
        vals = jnp.where(kcol == r, m, vals)
        ids = jnp.where(kcol == r, idx, ids)
        cand = jnp.where(candf == idx, jnp.float32(jnp.inf), cand)
        return cand, vals, ids

    _, vals, ids = jax.lax.fori_loop(
        0, K, exbody, (cand0, jnp.zeros((_QBB, K), jnp.float32),
                       jnp.zeros((_QBB, K), jnp.int32)))
    ids_ref[...] = ids
    vals_ref[...] = vals


def _topk_big(points, samples):
    s = samples.shape[0]
    spad = (-s) % _QBB
    sp = jnp.pad(samples, ((0, spad), (0, 0)))
    nq = sp.shape[0]
    n = points.shape[0]
    planes = jnp.broadcast_to(points.T[:, None, :], (3, _QBB, n))
    grid = nq // _QBB
    ids, vals = pl.pallas_call(
        _topk_big_body,
        grid=(grid,),
        in_specs=[
            pl.BlockSpec((_QBB, n), lambda i: (0, 0)),
            pl.BlockSpec((_QBB, n), lambda i: (0, 0)),
            pl.BlockSpec((_QBB, n), lambda i: (0, 0)),
            pl.BlockSpec((_QBB, 3), lambda i: (i, 0)),
        ],
        out_specs=[
            pl.BlockSpec((_QBB, K), lambda i: (i, 0)),
            pl.BlockSpec((_QBB, K), lambda i: (i, 0)),
        ],
        out_shape=[
            jax.ShapeDtypeStruct((nq, K), jnp.int32),
            jax.ShapeDtypeStruct((nq, K), jnp.float32),
        ],
        scratch_shapes=[
            pltpu.VMEM((_QBB, n), jnp.float32),
            pltpu.VMEM((_QBB, K * _G), jnp.float32),
            pltpu.VMEM((_QBB, K * _G), jnp.int32),
        ],
    )(planes[0], planes[1], planes[2], sp)
    return ids[:s], vals[:s]


def _topk(points, samples):
    s = samples.shape[0]
    spad = (-s) % _QB
    sp = jnp.pad(samples, ((0, spad), (0, 0)))
    nq = sp.shape[0]
    npad = (-points.shape[0]) % 128
    pp = jnp.pad(points, ((0, npad), (0, 0)), constant_values=1e6)
    n = pp.shape[0]
    planes = jnp.broadcast_to(pp.T[:, None, :], (3, _QB, n))
    grid = nq // _QB
    ids, vals = pl.pallas_call(
        _topk_body,
        grid=(grid,),
        in_specs=[
            pl.BlockSpec((_QB, n), lambda i: (0, 0)),
            pl.BlockSpec((_QB, n), lambda i: (0, 0)),
            pl.BlockSpec((_QB, n), lambda i: (0, 0)),
            pl.BlockSpec((_QB, 3), lambda i: (i, 0)),
        ],
        out_specs=[
            pl.BlockSpec((_QB, K), lambda i: (i, 0)),
            pl.BlockSpec((_QB, K), lambda i: (i, 0)),
        ],
        out_shape=[
            jax.ShapeDtypeStruct((nq, K), jnp.int32),
            jax.ShapeDtypeStruct((nq, K), jnp.float32),
        ],
    )(planes[0], planes[1], planes[2], sp)
    return ids[:s], vals[:s]


def kernel(points, W1, b1, W2, b2, W3, b3, W4, b4,
           Wd1, bd1, Wd2, bd2, Wd3, bd3, Wd4, bd4):
    s_inds = _fps(points, N1)
    samples = points[s_inds]

    ids1, dv1 = _topk_big(points, samples)
    id1 = ids1.reshape(-1)
    v1 = (dv1 <= R1 * R1).reshape(-1)
    rad_points = points[id1]
    midpoints = jnp.repeat(samples, K, axis=0)
    relative = (rad_points - midpoints) / R1 * v1[:, None].astype(points.dtype)

    s2_inds = _fps(samples, N2)
    samples2 = samples[s2_inds]

    ids2, dv2 = _topk(samples, samples2)
    id2 = ids2.reshape(-1)
    v2 = (dv2 <= R2 * R2).reshape(-1)
    rad2_points = samples[id2]
    midpoints2 = jnp.repeat(samples2, K, axis=0)
    relative2 = (rad2_points - midpoints2) / R2 * v2[:, None].astype(points.dtype)

    h1 = jax.nn.relu(relative @ W1 + b1)
    h1 = jax.nn.relu(h1 @ W2 + b2)
    feats = jnp.max(h1.reshape(N1, K, -1), axis=1)

    inp2 = jnp.concatenate([relative2, feats[id2]], axis=1)
    h2 = jax.nn.relu(inp2 @ W3 + b3)
    h2 = jax.nn.relu(h2 @ W4 + b4)
    encoding = jnp.max(h2.reshape(N2, K, -1), axis=1)

    gfeat = jnp.max(feats, axis=0)
    dec_in = jnp.concatenate([encoding, jnp.tile(gfeat[None, :], (N2, 1))], axis=1)
    d1 = jax.nn.relu(dec_in @ Wd1 + bd1)
    mid_feats = (d1 @ Wd2 + bd2).reshape(N2 * 20, 64)
    decoded = mid_feats @ Wd3 + bd3
    decoded2 = (mid_feats @ Wd4 + bd4).reshape(N2 * 20 * 20, 3)

    midpoints_out = (jnp.repeat(samples2, 20, axis=0) + decoded) * R2
    points_out = (jnp.repeat(midpoints_out, 20, axis=0) + decoded2) * R1
    return points_out
